# Initial kernel scaffold; baseline (speedup 1.0000x reference)
import jax
import jax.numpy as jnp
from jax.experimental import pallas as pl


def kernel(ingredient_x, taste_x, image_x, nutrient, caption, params, user_ids, item_ids, edge_it, edge_ti, edge_imi, edge_ini, edge_ui, label_edge):
    raise NotImplementedError("write your pallas kernel here")


# speedup vs baseline: 1.2796x; 1.2796x over previous
import functools

import jax
import jax.numpy as jnp
from jax import lax
from jax.experimental import pallas as pl
from jax.experimental.pallas import tpu as pltpu

HID = 256
ALPHA = 0.5
TEMP = 0.5
SLOPE = 0.2
N_NODE = 10000
BLK = 1000


def _row_specs(n_blocks, *widths):
    return [pl.BlockSpec((BLK, w), lambda i: (i, 0)) for w in widths]


def _full_spec(shape):
    return pl.BlockSpec(shape, lambda i: tuple(0 for _ in shape))


def _gelu(x):
    return jax.nn.gelu(x)


def _enc_body(image_x, taste_x, nutrient, caption, u_rows, i_rows,
              U_img, V_img, b_img, W_taste, b_taste,
              Wn1, bn1, Wn2, bn2, Wc1, bc1, Wc2, bc2,
              W_user, b_user, W_item, b_item,
              image_o, taste_o, na_o, ca_o, int_o, user_o, item_o):
    image_o[...] = (image_x[...] @ U_img[...]) @ V_img[...] + b_img[...]
    taste_o[...] = taste_x[...] @ W_taste[...] + b_taste[...]
    ne = jax.nn.relu(nutrient[...] @ Wn1[...] + bn1[...]) @ Wn2[...] + bn2[...]
    na_o[...] = ne / (jnp.sqrt(jnp.sum(ne * ne, axis=1, keepdims=True)) + 1e-8)
    ce = jax.nn.relu(caption[...] @ Wc1[...] + bc1[...]) @ Wc2[...] + bc2[...]
    ca = ce / (jnp.sqrt(jnp.sum(ce * ce, axis=1, keepdims=True)) + 1e-8)
    ca_o[...] = ca
    int_o[...] = _gelu(ca)
    user_o[...] = u_rows[...] @ W_user[...] + b_user[...]
    item_o[...] = i_rows[...] @ W_item[...] + b_item[...]


def _encoders(image_x, taste_x, nutrient, caption, u_rows, i_rows, p):
    n = image_x.shape[0]
    g = n // BLK
    f32 = jnp.float32
    w = lambda a: _full_spec(a.shape)
    args = (image_x, taste_x, nutrient, caption, u_rows, i_rows,
            p['U_img'], p['V_img'], p['b_img'], p['W_taste'], p['b_taste'],
            p['Wn1'], p['bn1'], p['Wn2'], p['bn2'],
            p['Wc1'], p['bc1'], p['Wc2'], p['bc2'],
            p['W_user'], p['b_user'], p['W_item'], p['b_item'])
    in_specs = (_row_specs(g, 768, 128, 32, 768, 64, 64)
                + [w(a) for a in args[6:]])
    out_specs = _row_specs(g, HID, HID, HID, HID, HID, HID, HID)
    outs = [jax.ShapeDtypeStruct((n, HID), f32) for _ in range(7)]
    return pl.pallas_call(
        _enc_body, grid=(g,), in_specs=in_specs, out_specs=out_specs,
        out_shape=outs)(*args)


def _ing_body(x, W, b, o):
    o[...] = x[...] @ W[...] + b[...]


def _ingredient_enc(ingredient_x, W, b):
    n = ingredient_x.shape[0]
    g = n // BLK
    return pl.pallas_call(
        _ing_body, grid=(g,),
        in_specs=[pl.BlockSpec((BLK, 128), lambda i: (i, 0)),
                  _full_spec(W.shape), _full_spec(b.shape)],
        out_specs=pl.BlockSpec((BLK, HID), lambda i: (i, 0)),
        out_shape=jax.ShapeDtypeStruct((n, HID), jnp.float32))(
            ingredient_x, W, b)


def _flash_body(ca_ref, na_ref, out_ref, rowl, coll, acc, *, nblk):
    i = pl.program_id(0)
    j = pl.program_id(1)
    s = lax.dot_general(ca_ref[...], na_ref[...], (((1,), (1,)), ((), ())),
                        preferred_element_type=jnp.float32) * (1.0 / TEMP)
    e = jnp.exp(s)
    rs = jnp.sum(e, axis=1, keepdims=True)
    cs = jnp.sum(e, axis=0, keepdims=True)

    @pl.when(jnp.logical_and(i == 0, j == 0))
    def _():
        acc[0] = 0.0
        acc[1] = 0.0

    @pl.when(j == 0)
    def _():
        rowl[...] = rs

    @pl.when(j > 0)
    def _():
        rowl[...] = rowl[...] + rs

    @pl.when(i == 0)
    def _():
        coll[pl.ds(j, 1), :] = cs

    @pl.when(i > 0)
    def _():
        coll[pl.ds(j, 1), :] = coll[pl.ds(j, 1), :] + cs

    @pl.when(i == j)
    def _():
        r = lax.broadcasted_iota(jnp.int32, (BLK, BLK), 0)
        c = lax.broadcasted_iota(jnp.int32, (BLK, BLK), 1)
        acc[0] = acc[0] + jnp.sum(jnp.where(r == c, s, 0.0))

    @pl.when(j == nblk - 1)
    def _():
        acc[1] = acc[1] + jnp.sum(jnp.log(rowl[...]))

    @pl.when(jnp.logical_and(i == nblk - 1, j == nblk - 1))
    def _():
        scol = jnp.sum(jnp.log(coll[...]))
        n = nblk * BLK
        val = (0.5 * acc[1] + 0.5 * scol - acc[0]) / n
        out_ref[...] = jnp.full((1, 1), val, jnp.float32)


def _flash_cl_loss(ca, na):
    n = ca.shape[0]
    g = n // BLK
    body = functools.partial(_flash_body, nblk=g)
    out = pl.pallas_call(
        body, grid=(g, g),
        in_specs=[pl.BlockSpec((BLK, HID), lambda i, j: (i, 0)),
                  pl.BlockSpec((BLK, HID), lambda i, j: (j, 0))],
        out_specs=pl.BlockSpec((1, 1), lambda i, j: (0, 0)),
        out_shape=jax.ShapeDtypeStruct((1, 1), jnp.float32),
        scratch_shapes=[pltpu.VMEM((BLK, 1), jnp.float32),
                        pltpu.VMEM((g, BLK), jnp.float32),
                        pltpu.SMEM((2,), jnp.float32)],
        compiler_params=pltpu.CompilerParams(
            dimension_semantics=("arbitrary", "arbitrary")))(ca, na)
    return out[0, 0]


def _taste_body(aggs, cnt, taste0, W, o):
    c = jnp.maximum(cnt[...], 1.0)
    o[...] = _gelu((aggs[...] / c) @ W[...] + 0.25 * taste0[...])


def _taste_update(aggs, cnt, taste0, Wcomb):
    n = taste0.shape[0]
    g = n // BLK
    return pl.pallas_call(
        _taste_body, grid=(g,),
        in_specs=[pl.BlockSpec((BLK, HID), lambda i: (i, 0)),
                  pl.BlockSpec((BLK, 1), lambda i: (i, 0)),
                  pl.BlockSpec((BLK, HID), lambda i: (i, 0)),
                  _full_spec(Wcomb.shape)],
        out_specs=pl.BlockSpec((BLK, HID), lambda i: (i, 0)),
        out_shape=jax.ShapeDtypeStruct((n, HID), jnp.float32))(
            aggs, cnt, taste0, Wcomb)


def _smix_body(aT, cT, aN, cN, aM, cM, w00, w01, w02, w10, w11, w12, s0, s1):
    mt = aT[...] / jnp.maximum(cT[...], 1.0)
    mn = aN[...] / jnp.maximum(cN[...], 1.0)
    mm = aM[...] / jnp.maximum(cM[...], 1.0)
    s0[...] = mt @ w00[...] + mn @ w01[...] + mm @ w02[...]
    s1[...] = mt @ w10[...] + mn @ w11[...] + mm @ w12[...]


def _static_messages(aT, cT, aN, cN, aM, cM, ws):
    n = aT.shape[0]
    g = n // BLK
    rs = lambda w_: pl.BlockSpec((BLK, w_), lambda i: (i, 0))
    return pl.pallas_call(
        _smix_body, grid=(g,),
        in_specs=[rs(HID), rs(1), rs(HID), rs(1), rs(HID), rs(1)]
        + [_full_spec((HID, HID))] * 6,
        out_specs=[rs(HID), rs(HID)],
        out_shape=[jax.ShapeDtypeStruct((n, HID), jnp.float32)] * 2)(
            aT, cT, aN, cN, aM, cM, *ws)


def _layer_body(S, aU, cU, aV, cV, user, item, W3, W4, item_o, user_o, *,
                final):
    mu = (aU[...] / jnp.maximum(cU[...], 1.0)) @ W3[...]
    un = (aV[...] / jnp.maximum(cV[...], 1.0)) @ W4[...]
    it = (S[...] + mu) * 0.125 + 0.5 * item[...]
    us = 0.5 * un + 0.5 * user[...]
    if final:
        it = _gelu(it)
        us = _gelu(us)
    item_o[...] = it
    user_o[...] = us


def _fusion_layer(S, aU, cU, aV, cV, user, item, W3, W4, final):
    n = user.shape[0]
    g = n // BLK
    rs = lambda w_: pl.BlockSpec((BLK, w_), lambda i: (i, 0))
    body = functools.partial(_layer_body, final=final)
    return pl.pallas_call(
        body, grid=(g,),
        in_specs=[rs(HID), rs(HID), rs(1), rs(HID), rs(1), rs(HID), rs(HID),
                  _full_spec((HID, HID)), _full_spec((HID, HID))],
        out_specs=[rs(HID), rs(HID)],
        out_shape=[jax.ShapeDtypeStruct((n, HID), jnp.float32)] * 2)(
            S, aU, cU, aV, cV, user, item, W3, W4)


def _lp_body(gu, gi, W1a, W1b, b1, W2, b2, o):
    h = gu[...] @ W1a[...] + gi[...] @ W1b[...] + b1[...]
    h = jnp.where(h > 0, h, SLOPE * h)
    o[...] = h @ W2[...] + b2[...]


def _link_pred(gu, gi, W1a, W1b, b1, W2, b2):
    n = gu.shape[0]
    g = n // BLK
    rs = lambda w_: pl.BlockSpec((BLK, w_), lambda i: (i, 0))
    return pl.pallas_call(
        _lp_body, grid=(g,),
        in_specs=[rs(HID), rs(HID), _full_spec((HID, HID)),
                  _full_spec((HID, HID)), _full_spec((1, HID)),
                  _full_spec((HID, 1)), _full_spec((1, 1))],
        out_specs=rs(1),
        out_shape=jax.ShapeDtypeStruct((n, 1), jnp.float32))(
            gu, gi, W1a, W1b, b1, W2, b2)


def _seg_sum_cnt(vals, idx, n):
    s = jax.ops.segment_sum(vals, idx, num_segments=n)
    c = jax.ops.segment_sum(jnp.ones((vals.shape[0], 1), vals.dtype), idx,
                            num_segments=n)
    return s, c


def _gather_rows(table, ids):
    return jnp.take(table, ids, axis=0)


def kernel(ingredient_x, taste_x, image_x, nutrient, caption, params,
           user_ids, item_ids, edge_it, edge_ti, edge_imi, edge_ini,
           edge_ui, label_edge):
    p = params
    f32 = jnp.float32
    row = lambda b: b.reshape(1, -1).astype(f32)

    u_rows = _gather_rows(p['user_table'], user_ids)
    i_rows = _gather_rows(p['item_table'], item_ids)

    image, taste0, na, ca, intention, user0, item0 = _encoders(
        image_x, taste_x, nutrient, caption, u_rows, i_rows,
        {**p, 'b_img': row(p['b_img']), 'b_taste': row(p['b_taste']),
         'bn1': row(p['bn1']), 'bn2': row(p['bn2']), 'bc1': row(p['bc1']),
         'bc2': row(p['bc2']), 'b_user': row(p['b_user']),
         'b_item': row(p['b_item'])})
    ing0 = _ingredient_enc(ingredient_x, p['W_ing'], row(p['b_ing']))

    cl_loss = _flash_cl_loss(ca, na)

    aggI, cntI = _seg_sum_cnt(ing0[edge_it[0]], edge_it[1], N_NODE)
    Wcomb = 0.5 * p['W_sense'][1] + 0.25 * p['W_sense'][0]
    taste_f = _taste_update(aggI, cntI, taste0, Wcomb)

    aggT, cntT = _seg_sum_cnt(taste_f[edge_ti[0]], edge_ti[1], N_NODE)
    aggN, cntN = _seg_sum_cnt(intention[edge_ini[0]], edge_ini[1], N_NODE)
    aggM, cntM = _seg_sum_cnt(image[edge_imi[0]], edge_imi[1], N_NODE)
    Wf = p['W_fus']
    S0, S1 = _static_messages(
        aggT, cntT, aggN, cntN, aggM, cntM,
        [Wf[0, 0], Wf[0, 1], Wf[0, 2], Wf[1, 0], Wf[1, 1], Wf[1, 2]])

    user, item = user0, item0
    for l, S in ((0, S0), (1, S1)):
        aggU, cntU = _seg_sum_cnt(user[edge_ui[0]], edge_ui[1], N_NODE)
        aggV, cntV = _seg_sum_cnt(item[edge_ui[1]], edge_ui[0], N_NODE)
        item, user = _fusion_layer(S, aggU, cntU, aggV, cntV, user, item,
                                   Wf[l, 3], Wf[l, 4], final=(l == 1))

    gu = _gather_rows(user, label_edge[0])
    gi = _gather_rows(item, label_edge[1])
    scores = _link_pred(gu, gi, p['W_lp1'][:HID], p['W_lp1'][HID:],
                        row(p['b_lp1']), p['W_lp2'], row(p['b_lp2']))
    return (scores, cl_loss)


# speedup vs baseline: 2.6320x; 2.0568x over previous
import functools

import jax
import jax.numpy as jnp
from jax import lax
from jax.experimental import pallas as pl
from jax.experimental.pallas import tpu as pltpu
from jax.experimental.pallas import tpu_sc as plsc

HID = 256
ALPHA = 0.5
TEMP = 0.5
SLOPE = 0.2
N_NODE = 10000
BLK = 1000


def _row_specs(n_blocks, *widths):
    return [pl.BlockSpec((BLK, w), lambda i: (i, 0)) for w in widths]


def _full_spec(shape):
    return pl.BlockSpec(shape, lambda i: tuple(0 for _ in shape))


def _gelu(x):
    return jax.nn.gelu(x)


def _enc_body(image_x, taste_x, nutrient, caption, u_rows, i_rows,
              U_img, V_img, b_img, W_taste, b_taste,
              Wn1, bn1, Wn2, bn2, Wc1, bc1, Wc2, bc2,
              W_user, b_user, W_item, b_item,
              image_o, taste_o, na_o, ca_o, int_o, user_o, item_o):
    image_o[...] = (image_x[...] @ U_img[...]) @ V_img[...] + b_img[...]
    taste_o[...] = taste_x[...] @ W_taste[...] + b_taste[...]
    ne = jax.nn.relu(nutrient[...] @ Wn1[...] + bn1[...]) @ Wn2[...] + bn2[...]
    na_o[...] = ne / (jnp.sqrt(jnp.sum(ne * ne, axis=1, keepdims=True)) + 1e-8)
    ce = jax.nn.relu(caption[...] @ Wc1[...] + bc1[...]) @ Wc2[...] + bc2[...]
    ca = ce / (jnp.sqrt(jnp.sum(ce * ce, axis=1, keepdims=True)) + 1e-8)
    ca_o[...] = ca
    int_o[...] = _gelu(ca)
    user_o[...] = u_rows[...] @ W_user[...] + b_user[...]
    item_o[...] = i_rows[...] @ W_item[...] + b_item[...]


def _encoders(image_x, taste_x, nutrient, caption, u_rows, i_rows, p):
    n = image_x.shape[0]
    g = n // BLK
    f32 = jnp.float32
    w = lambda a: _full_spec(a.shape)
    args = (image_x, taste_x, nutrient, caption, u_rows, i_rows,
            p['U_img'], p['V_img'], p['b_img'], p['W_taste'], p['b_taste'],
            p['Wn1'], p['bn1'], p['Wn2'], p['bn2'],
            p['Wc1'], p['bc1'], p['Wc2'], p['bc2'],
            p['W_user'], p['b_user'], p['W_item'], p['b_item'])
    in_specs = (_row_specs(g, 768, 128, 32, 768, 64, 64)
                + [w(a) for a in args[6:]])
    out_specs = _row_specs(g, HID, HID, HID, HID, HID, HID, HID)
    outs = [jax.ShapeDtypeStruct((n, HID), f32) for _ in range(7)]
    return pl.pallas_call(
        _enc_body, grid=(g,), in_specs=in_specs, out_specs=out_specs,
        out_shape=outs)(*args)


def _ing_body(x, W, b, o):
    o[...] = x[...] @ W[...] + b[...]


def _ingredient_enc(ingredient_x, W, b):
    n = ingredient_x.shape[0]
    g = n // BLK
    return pl.pallas_call(
        _ing_body, grid=(g,),
        in_specs=[pl.BlockSpec((BLK, 128), lambda i: (i, 0)),
                  _full_spec(W.shape), _full_spec(b.shape)],
        out_specs=pl.BlockSpec((BLK, HID), lambda i: (i, 0)),
        out_shape=jax.ShapeDtypeStruct((n, HID), jnp.float32))(
            ingredient_x, W, b)


def _flash_body(ca_ref, na_ref, out_ref, rowl, coll, acc, *, nblk):
    i = pl.program_id(0)
    j = pl.program_id(1)
    s = lax.dot_general(ca_ref[...], na_ref[...], (((1,), (1,)), ((), ())),
                        preferred_element_type=jnp.float32) * (1.0 / TEMP)
    e = jnp.exp(s)
    rs = jnp.sum(e, axis=1, keepdims=True)
    cs = jnp.sum(e, axis=0, keepdims=True)

    @pl.when(jnp.logical_and(i == 0, j == 0))
    def _():
        acc[0] = 0.0
        acc[1] = 0.0

    @pl.when(j == 0)
    def _():
        rowl[...] = rs

    @pl.when(j > 0)
    def _():
        rowl[...] = rowl[...] + rs

    @pl.when(i == 0)
    def _():
        coll[pl.ds(j, 1), :] = cs

    @pl.when(i > 0)
    def _():
        coll[pl.ds(j, 1), :] = coll[pl.ds(j, 1), :] + cs

    @pl.when(i == j)
    def _():
        r = lax.broadcasted_iota(jnp.int32, (BLK, BLK), 0)
        c = lax.broadcasted_iota(jnp.int32, (BLK, BLK), 1)
        acc[0] = acc[0] + jnp.sum(jnp.where(r == c, s, 0.0))

    @pl.when(j == nblk - 1)
    def _():
        acc[1] = acc[1] + jnp.sum(jnp.log(rowl[...]))

    @pl.when(jnp.logical_and(i == nblk - 1, j == nblk - 1))
    def _():
        scol = jnp.sum(jnp.log(coll[...]))
        n = nblk * BLK
        val = (0.5 * acc[1] + 0.5 * scol - acc[0]) / n
        out_ref[...] = jnp.full((1, 1), val, jnp.float32)


def _flash_cl_loss(ca, na):
    n = ca.shape[0]
    g = n // BLK
    body = functools.partial(_flash_body, nblk=g)
    out = pl.pallas_call(
        body, grid=(g, g),
        in_specs=[pl.BlockSpec((BLK, HID), lambda i, j: (i, 0)),
                  pl.BlockSpec((BLK, HID), lambda i, j: (j, 0))],
        out_specs=pl.BlockSpec((1, 1), lambda i, j: (0, 0)),
        out_shape=jax.ShapeDtypeStruct((1, 1), jnp.float32),
        scratch_shapes=[pltpu.VMEM((BLK, 1), jnp.float32),
                        pltpu.VMEM((g, BLK), jnp.float32),
                        pltpu.SMEM((2,), jnp.float32)],
        compiler_params=pltpu.CompilerParams(
            dimension_semantics=("arbitrary", "arbitrary")))(ca, na)
    return out[0, 0]


def _taste_body(aggs, cnt, taste0, W, o):
    c = jnp.maximum(cnt[...], 1.0)
    o[...] = _gelu((aggs[...] / c) @ W[...] + 0.25 * taste0[...])


def _taste_update(aggs, cnt, taste0, Wcomb):
    n = taste0.shape[0]
    g = n // BLK
    return pl.pallas_call(
        _taste_body, grid=(g,),
        in_specs=[pl.BlockSpec((BLK, HID), lambda i: (i, 0)),
                  pl.BlockSpec((BLK, 1), lambda i: (i, 0)),
                  pl.BlockSpec((BLK, HID), lambda i: (i, 0)),
                  _full_spec(Wcomb.shape)],
        out_specs=pl.BlockSpec((BLK, HID), lambda i: (i, 0)),
        out_shape=jax.ShapeDtypeStruct((n, HID), jnp.float32))(
            aggs, cnt, taste0, Wcomb)


def _smix_body(aT, cT, aN, cN, aM, cM, w00, w01, w02, w10, w11, w12, s0, s1):
    mt = aT[...] / jnp.maximum(cT[...], 1.0)
    mn = aN[...] / jnp.maximum(cN[...], 1.0)
    mm = aM[...] / jnp.maximum(cM[...], 1.0)
    s0[...] = mt @ w00[...] + mn @ w01[...] + mm @ w02[...]
    s1[...] = mt @ w10[...] + mn @ w11[...] + mm @ w12[...]


def _static_messages(aT, cT, aN, cN, aM, cM, ws):
    n = aT.shape[0]
    g = n // BLK
    rs = lambda w_: pl.BlockSpec((BLK, w_), lambda i: (i, 0))
    return pl.pallas_call(
        _smix_body, grid=(g,),
        in_specs=[rs(HID), rs(1), rs(HID), rs(1), rs(HID), rs(1)]
        + [_full_spec((HID, HID))] * 6,
        out_specs=[rs(HID), rs(HID)],
        out_shape=[jax.ShapeDtypeStruct((n, HID), jnp.float32)] * 2)(
            aT, cT, aN, cN, aM, cM, *ws)


def _layer_body(S, aU, cU, aV, cV, user, item, W3, W4, item_o, user_o, *,
                final):
    mu = (aU[...] / jnp.maximum(cU[...], 1.0)) @ W3[...]
    un = (aV[...] / jnp.maximum(cV[...], 1.0)) @ W4[...]
    it = (S[...] + mu) * 0.125 + 0.5 * item[...]
    us = 0.5 * un + 0.5 * user[...]
    if final:
        it = _gelu(it)
        us = _gelu(us)
    item_o[...] = it
    user_o[...] = us


def _fusion_layer(S, aU, cU, aV, cV, user, item, W3, W4, final):
    n = user.shape[0]
    g = n // BLK
    rs = lambda w_: pl.BlockSpec((BLK, w_), lambda i: (i, 0))
    body = functools.partial(_layer_body, final=final)
    return pl.pallas_call(
        body, grid=(g,),
        in_specs=[rs(HID), rs(HID), rs(1), rs(HID), rs(1), rs(HID), rs(HID),
                  _full_spec((HID, HID)), _full_spec((HID, HID))],
        out_specs=[rs(HID), rs(HID)],
        out_shape=[jax.ShapeDtypeStruct((n, HID), jnp.float32)] * 2)(
            S, aU, cU, aV, cV, user, item, W3, W4)


def _lp_body(gu, gi, W1a, W1b, b1, W2, b2, o):
    h = gu[...] @ W1a[...] + gi[...] @ W1b[...] + b1[...]
    h = jnp.where(h > 0, h, SLOPE * h)
    o[...] = h @ W2[...] + b2[...]


def _link_pred(gu, gi, W1a, W1b, b1, W2, b2):
    n = gu.shape[0]
    g = n // BLK
    rs = lambda w_: pl.BlockSpec((BLK, w_), lambda i: (i, 0))
    return pl.pallas_call(
        _lp_body, grid=(g,),
        in_specs=[rs(HID), rs(HID), _full_spec((HID, HID)),
                  _full_spec((HID, HID)), _full_spec((1, HID)),
                  _full_spec((HID, 1)), _full_spec((1, 1))],
        out_specs=rs(1),
        out_shape=jax.ShapeDtypeStruct((n, 1), jnp.float32))(
            gu, gi, W1a, W1b, b1, W2, b2)


NC, NS, LANES = 2, 16, 16
MP = 10240
CH = 128
ROWS_PER_TILE = MP // NS


def _sc_segsum(tab_all, src2, dst, zf):
    epad = dst.shape[0]
    n_chunks = epad // (NS * CH)
    f32 = jnp.float32
    vmesh = plsc.VectorSubcoreMesh(core_axis_name="c", subcore_axis_name="s")

    @functools.partial(
        pl.kernel,
        out_type=jax.ShapeDtypeStruct((NC * MP, 128), f32),
        mesh=vmesh,
        scratch_types=[pltpu.VMEM((CH,), jnp.int32),
                       pltpu.VMEM((CH,), jnp.int32),
                       pltpu.VMEM((CH, 128), f32),
                       pltpu.VMEM((ROWS_PER_TILE // 4, 128), f32),
                       pltpu.VMEM_SHARED((MP, 128), f32),
                       pltpu.SemaphoreType.DMA],
    )
    def k(ta, src_h, dst_h, zf_h, feat_o,
          idx_s, idx_d, rows, stg_f, acc_f, sem):
        c = lax.axis_index("c")
        s = lax.axis_index("s")
        base_row = s * ROWS_PER_TILE
        q = ROWS_PER_TILE // 4
        pltpu.sync_copy(zf_h.at[pl.ds(0, q)], stg_f)
        for piece in range(4):
            pltpu.sync_copy(stg_f, acc_f.at[pl.ds(base_row + piece * q, q)])
        plsc.subcore_barrier()

        def chunk(g, _):
            base = (s * n_chunks + g) * CH
            pltpu.sync_copy(src_h.at[pl.ds(c * epad + base, CH)], idx_s)
            pltpu.sync_copy(dst_h.at[pl.ds(base, CH)], idx_d)
            pltpu.async_copy(ta.at[idx_s], rows, sem).wait()
            pltpu.sync_copy(rows, acc_f.at[idx_d], add=True)
            return 0

        lax.fori_loop(0, n_chunks, chunk, 0)

        plsc.subcore_barrier()
        obase = c * MP + base_row
        for piece in range(4):
            pltpu.sync_copy(acc_f.at[pl.ds(base_row + piece * q, q)], stg_f)
            pltpu.sync_copy(stg_f,
                            feat_o.at[pl.ds(obase + piece * q, q)])

    feat_o = k(tab_all, src2, dst, zf)
    return feat_o.reshape(NC, MP, 128)


def _seg_sum_cnt(vals, src, dst, aux, tok):
    epad_unit = NS * CH
    e = src.shape[0]
    epad = ((e + epad_unit - 1) // epad_unit) * epad_unit
    src_p = jnp.concatenate([src, jnp.zeros((epad - e,), jnp.int32)])
    dst_p = jnp.concatenate(
        [dst, jnp.full((epad - e,), N_NODE, jnp.int32)])
    n = vals.shape[0]
    tab_all = jnp.concatenate([vals[:, :128], vals[:, 128:]], axis=0)
    src2 = jnp.concatenate([src_p, src_p + n])
    zf = aux[0]
    feat2 = _sc_segsum(tab_all, src2, dst_p, zf + tok)
    feat = jnp.concatenate([feat2[0, :N_NODE], feat2[1, :N_NODE]], axis=1)
    cnt = jax.ops.segment_sum(jnp.ones((e, 1), jnp.float32), dst,
                              num_segments=N_NODE)
    return feat, cnt, _tok0(feat2)


def _tok0(x):
    return x.ravel()[0] * 0.0


GCH = 80


def _sc_gather(table, ids, tok):
    v, d = table.shape
    b = ids.shape[0]
    bp = 10240
    ids_p = jnp.concatenate([ids.astype(jnp.int32),
                             jnp.zeros((bp - b,), jnp.int32)
                             + tok.astype(jnp.int32)])
    b_per_w = bp // (NC * NS)
    n_chunks = b_per_w // GCH
    vmesh = plsc.VectorSubcoreMesh(core_axis_name="c", subcore_axis_name="s")

    @functools.partial(
        pl.kernel,
        out_type=jax.ShapeDtypeStruct((bp, d), jnp.float32),
        mesh=vmesh,
        scratch_types=[pltpu.VMEM((GCH,), jnp.int32),
                       pltpu.VMEM((GCH, d), jnp.float32),
                       pltpu.SemaphoreType.DMA],
    )
    def k(tab, ids_h, out_h, idx_v, rows, sem):
        c = lax.axis_index("c")
        s = lax.axis_index("s")
        wid = s * NC + c

        def chunk(g, _):
            base = wid * b_per_w + g * GCH
            pltpu.sync_copy(ids_h.at[pl.ds(base, GCH)], idx_v)
            pltpu.async_copy(tab.at[idx_v], rows, sem).wait()
            pltpu.sync_copy(rows, out_h.at[pl.ds(base, GCH)])
            return 0

        lax.fori_loop(0, n_chunks, chunk, 0)

    out = k(table, ids_p)
    return out[:b], _tok0(out)


def kernel(ingredient_x, taste_x, image_x, nutrient, caption, params,
           user_ids, item_ids, edge_it, edge_ti, edge_imi, edge_ini,
           edge_ui, label_edge):
    p = params
    f32 = jnp.float32
    row = lambda b: b.reshape(1, -1).astype(f32)
    aux = (jnp.zeros((ROWS_PER_TILE // 4, 128), f32),)

    pad64 = lambda t: jnp.pad(t, ((0, 0), (0, 64)))
    tok = jnp.float32(0.0)
    u_rows, tok = _sc_gather(pad64(p['user_table']), user_ids, tok)
    i_rows, tok = _sc_gather(pad64(p['item_table']), item_ids, tok)
    u_rows, i_rows = u_rows[:, :64], i_rows[:, :64]

    image, taste0, na, ca, intention, user0, item0 = _encoders(
        image_x, taste_x, nutrient, caption, u_rows, i_rows,
        {**p, 'b_img': row(p['b_img']), 'b_taste': row(p['b_taste']),
         'bn1': row(p['bn1']), 'bn2': row(p['bn2']), 'bc1': row(p['bc1']),
         'bc2': row(p['bc2']), 'b_user': row(p['b_user']),
         'b_item': row(p['b_item'])})
    ing0 = _ingredient_enc(ingredient_x, p['W_ing'], row(p['b_ing']))

    cl_loss = _flash_cl_loss(ca, na)

    aggI, cntI, tok = _seg_sum_cnt(ing0, edge_it[0], edge_it[1], aux, tok)
    Wcomb = 0.5 * p['W_sense'][1] + 0.25 * p['W_sense'][0]
    taste_f = _taste_update(aggI, cntI, taste0, Wcomb)

    aggT, cntT, tok = _seg_sum_cnt(taste_f, edge_ti[0], edge_ti[1], aux, tok)
    aggN, cntN, tok = _seg_sum_cnt(intention, edge_ini[0], edge_ini[1], aux,
                                   tok)
    aggM, cntM, tok = _seg_sum_cnt(image, edge_imi[0], edge_imi[1], aux, tok)
    Wf = p['W_fus']
    S0, S1 = _static_messages(
        aggT, cntT, aggN, cntN, aggM, cntM,
        [Wf[0, 0], Wf[0, 1], Wf[0, 2], Wf[1, 0], Wf[1, 1], Wf[1, 2]])

    user, item = user0, item0
    cntU = cntV = None
    for l, S in ((0, S0), (1, S1)):
        aggU, cU, tok = _seg_sum_cnt(user, edge_ui[0], edge_ui[1], aux, tok)
        aggV, cV, tok = _seg_sum_cnt(item, edge_ui[1], edge_ui[0], aux, tok)
        cntU = cU if cntU is None else cntU
        cntV = cV if cntV is None else cntV
        item, user = _fusion_layer(S, aggU, cntU, aggV, cntV, user, item,
                                   Wf[l, 3], Wf[l, 4], final=(l == 1))

    gu, tok = _sc_gather(user, label_edge[0], tok)
    gi, tok = _sc_gather(item, label_edge[1], tok)
    scores = _link_pred(gu, gi, p['W_lp1'][:HID], p['W_lp1'][HID:],
                        row(p['b_lp1']), p['W_lp2'], row(p['b_lp2']))
    return (scores, cl_loss)
